# baseline (speedup 1.0000x reference)
import jax
import jax.numpy as jnp
from jax import lax
from jax.experimental import pallas as pl

NUM_CODES = 1024
DIM = 64
PIX = 1024


BATCH_BLOCK = 1


def _vq_kernel(x_ref, cb_ref, idx_ref, q_ref):
    cb = cb_ref[...]
    cnorm = jnp.sum(cb * cb, axis=1, keepdims=True)

    for i in range(BATCH_BLOCK):
        xb = x_ref[i]

        xsq = xb * xb
        xnorm = jnp.sum(xsq.T, axis=1, keepdims=True).T
        cross = lax.dot_general(cb, xb, (((1,), (0,)), ((), ())),
                                preferred_element_type=jnp.float32)
        dist = (xnorm + cnorm) - 2.0 * cross

        minval = jnp.min(dist, axis=0, keepdims=True)
        rowid = lax.broadcasted_iota(jnp.int32, dist.shape, 0)
        idx = jnp.min(jnp.where(dist == minval, rowid, NUM_CODES),
                      axis=0, keepdims=True)

        onehot = (rowid == idx).astype(jnp.float32)
        q = lax.dot_general(cb, onehot, (((0,), (0,)), ((), ())),
                            precision=lax.Precision.HIGHEST,
                            preferred_element_type=jnp.float32)

        idx_ref[i] = idx
        q_ref[i] = q


def kernel(x, codebook):
    B, C, H, W = x.shape
    xflat = x.reshape(B, C, H * W)
    nb = B // BATCH_BLOCK
    idx, q = pl.pallas_call(
        _vq_kernel,
        grid=(nb,),
        in_specs=[
            pl.BlockSpec((BATCH_BLOCK, C, H * W), lambda b: (b, 0, 0)),
            pl.BlockSpec((NUM_CODES, DIM), lambda b: (0, 0)),
        ],
        out_specs=[
            pl.BlockSpec((BATCH_BLOCK, 1, H * W), lambda b: (b, 0, 0)),
            pl.BlockSpec((BATCH_BLOCK, C, H * W), lambda b: (b, 0, 0)),
        ],
        out_shape=[
            jax.ShapeDtypeStruct((B, 1, H * W), jnp.int32),
            jax.ShapeDtypeStruct((B, C, H * W), jnp.float32),
        ],
    )(xflat, codebook)
    indices = idx.reshape(B * H * W, 1)
    quantized = q.reshape(B, C, H, W)
    return (indices, quantized)


# speedup vs baseline: 1.7142x; 1.7142x over previous
import jax
import jax.numpy as jnp
from jax import lax
from jax.experimental import pallas as pl

NUM_CODES = 1024
DIM = 64
PIX = 1024


BATCH_BLOCK = 2


def _vq_kernel(x_ref, cb_ref, idx_ref, q_ref):
    cb = cb_ref[...]
    cnorm = jnp.sum(cb * cb, axis=1, keepdims=True)

    for i in range(BATCH_BLOCK):
        xb = x_ref[i]

        xsq = xb * xb
        xnorm = jnp.sum(xsq.T, axis=1, keepdims=True).T
        cross = lax.dot_general(cb, xb, (((1,), (0,)), ((), ())),
                                preferred_element_type=jnp.float32)
        dist = (xnorm + cnorm) - 2.0 * cross

        minval = jnp.min(dist, axis=0, keepdims=True)
        rowid = lax.broadcasted_iota(jnp.int32, dist.shape, 0)
        idx = jnp.min(jnp.where(dist == minval, rowid, NUM_CODES),
                      axis=0, keepdims=True)

        onehot = (rowid == idx).astype(jnp.float32)
        q = lax.dot_general(cb, onehot, (((0,), (0,)), ((), ())),
                            preferred_element_type=jnp.float32)

        idx_ref[i] = idx
        q_ref[i] = q


def kernel(x, codebook):
    B, C, H, W = x.shape
    xflat = x.reshape(B, C, H * W)
    nb = B // BATCH_BLOCK
    idx, q = pl.pallas_call(
        _vq_kernel,
        grid=(nb,),
        in_specs=[
            pl.BlockSpec((BATCH_BLOCK, C, H * W), lambda b: (b, 0, 0)),
            pl.BlockSpec((NUM_CODES, DIM), lambda b: (0, 0)),
        ],
        out_specs=[
            pl.BlockSpec((BATCH_BLOCK, 1, H * W), lambda b: (b, 0, 0)),
            pl.BlockSpec((BATCH_BLOCK, C, H * W), lambda b: (b, 0, 0)),
        ],
        out_shape=[
            jax.ShapeDtypeStruct((B, 1, H * W), jnp.int32),
            jax.ShapeDtypeStruct((B, C, H * W), jnp.float32),
        ],
    )(xflat, codebook)
    indices = idx.reshape(B * H * W, 1)
    quantized = q.reshape(B, C, H, W)
    return (indices, quantized)


# speedup vs baseline: 1.7568x; 1.0249x over previous
import jax
import jax.numpy as jnp
from jax import lax
from jax.experimental import pallas as pl

NUM_CODES = 1024
DIM = 64
PIX = 1024


BATCH_BLOCK = 4


def _vq_kernel(x_ref, cb_ref, idx_ref, q_ref):
    cb = cb_ref[...]
    cnorm = jnp.sum(cb * cb, axis=1, keepdims=True)

    for i in range(BATCH_BLOCK):
        xb = x_ref[i]

        xsq = xb * xb
        xnorm = jnp.sum(xsq.T, axis=1, keepdims=True).T
        cross = lax.dot_general(cb, xb, (((1,), (0,)), ((), ())),
                                preferred_element_type=jnp.float32)
        dist = (xnorm + cnorm) - 2.0 * cross

        minval = jnp.min(dist, axis=0, keepdims=True)
        rowid = lax.broadcasted_iota(jnp.int32, dist.shape, 0)
        idx = jnp.min(jnp.where(dist == minval, rowid, NUM_CODES),
                      axis=0, keepdims=True)

        onehot = (rowid == idx).astype(jnp.float32)
        q = lax.dot_general(cb, onehot, (((0,), (0,)), ((), ())),
                            preferred_element_type=jnp.float32)

        idx_ref[i] = idx
        q_ref[i] = q


def kernel(x, codebook):
    B, C, H, W = x.shape
    xflat = x.reshape(B, C, H * W)
    nb = B // BATCH_BLOCK
    idx, q = pl.pallas_call(
        _vq_kernel,
        grid=(nb,),
        in_specs=[
            pl.BlockSpec((BATCH_BLOCK, C, H * W), lambda b: (b, 0, 0)),
            pl.BlockSpec((NUM_CODES, DIM), lambda b: (0, 0)),
        ],
        out_specs=[
            pl.BlockSpec((BATCH_BLOCK, 1, H * W), lambda b: (b, 0, 0)),
            pl.BlockSpec((BATCH_BLOCK, C, H * W), lambda b: (b, 0, 0)),
        ],
        out_shape=[
            jax.ShapeDtypeStruct((B, 1, H * W), jnp.int32),
            jax.ShapeDtypeStruct((B, C, H * W), jnp.float32),
        ],
    )(xflat, codebook)
    indices = idx.reshape(B * H * W, 1)
    quantized = q.reshape(B, C, H, W)
    return (indices, quantized)


# speedup vs baseline: 1.8283x; 1.0407x over previous
import jax
import jax.numpy as jnp
from jax import lax
from jax.experimental import pallas as pl

NUM_CODES = 1024
DIM = 64
PIX = 1024


BATCH_BLOCK = 4


def _vq_kernel(x_ref, cb_ref, idx_ref, q_ref):
    cb = cb_ref[...]
    cnorm = jnp.sum(cb * cb, axis=1, keepdims=True)
    cb2 = cb + cb

    for i in range(BATCH_BLOCK):
        xb = x_ref[i]

        xsq = xb * xb
        xnorm = jnp.sum(xsq.T, axis=1, keepdims=True).T
        cross2 = lax.dot_general(cb2, xb, (((1,), (0,)), ((), ())),
                                 preferred_element_type=jnp.float32)
        dist = (xnorm + cnorm) - cross2

        minval = jnp.min(dist, axis=0, keepdims=True)
        rowid = lax.broadcasted_iota(jnp.int32, dist.shape, 0)
        idx = jnp.min(jnp.where(dist == minval, rowid, NUM_CODES),
                      axis=0, keepdims=True)

        onehot = (rowid == idx).astype(jnp.float32)
        q = lax.dot_general(cb, onehot, (((0,), (0,)), ((), ())),
                            preferred_element_type=jnp.float32)

        idx_ref[i] = idx
        q_ref[i] = q


def kernel(x, codebook):
    B, C, H, W = x.shape
    xflat = x.reshape(B, C, H * W)
    nb = B // BATCH_BLOCK
    idx, q = pl.pallas_call(
        _vq_kernel,
        grid=(nb,),
        in_specs=[
            pl.BlockSpec((BATCH_BLOCK, C, H * W), lambda b: (b, 0, 0)),
            pl.BlockSpec((NUM_CODES, DIM), lambda b: (0, 0)),
        ],
        out_specs=[
            pl.BlockSpec((BATCH_BLOCK, 1, H * W), lambda b: (b, 0, 0)),
            pl.BlockSpec((BATCH_BLOCK, C, H * W), lambda b: (b, 0, 0)),
        ],
        out_shape=[
            jax.ShapeDtypeStruct((B, 1, H * W), jnp.int32),
            jax.ShapeDtypeStruct((B, C, H * W), jnp.float32),
        ],
    )(xflat, codebook)
    indices = idx.reshape(B * H * W, 1)
    quantized = q.reshape(B, C, H, W)
    return (indices, quantized)
